# Initial kernel scaffold; baseline (speedup 1.0000x reference)
import jax
import jax.numpy as jnp
from jax.experimental import pallas as pl


def kernel(token_ids, lang_ids, W_tok, W_lang, W_proj):
    raise NotImplementedError("write your pallas kernel here")


# speedup vs baseline: 1.1894x; 1.1894x over previous
import functools
import math

import jax
import jax.numpy as jnp
import numpy as np
from jax import lax
from jax.experimental import pallas as pl
from jax.experimental.pallas import tpu as pltpu
from jax.experimental.pallas import tpu_sc as plsc

VOCAB = 100000
D_MODEL = 768
NUM_LANG = 4
MAX_LEN = 4096
B = 4
S = 4096
SCALE = math.sqrt(D_MODEL)

_NW = 32
_SPW = S // _NW
_K = 32
_NSC = _SPW // _K
_L = 16
_NJ = D_MODEL // _L
_JB = 8


def _pe_np():
    pos = np.arange(MAX_LEN, dtype=np.float32)[:, None]
    div = np.exp(
        np.arange(0, D_MODEL, 2, dtype=np.float32)
        * np.float32(-math.log(10000.0) / D_MODEL)
    ).astype(np.float32)
    pe = np.zeros((MAX_LEN, D_MODEL), dtype=np.float32)
    pe[:, 0::2] = np.sin(pos * div)
    pe[:, 1::2] = np.cos(pos * div)
    return pe


_PE = _pe_np()
_GATHER_DN = lax.GatherDimensionNumbers(
    offset_dims=(), collapsed_slice_dims=(0,), start_index_map=(0,)
)


def _lane_splat(vec, lane):
    idx = jnp.full((16, 1), lane, jnp.int32)
    return lax.gather(
        vec, idx, _GATHER_DN, slice_sizes=(1,),
        mode=lax.GatherScatterMode.PROMISE_IN_BOUNDS,
    )


def _lang_tab_body(wl_ref, wp_ref, out_ref):
    out_ref[...] = lax.dot_general(
        wl_ref[...],
        wp_ref[...],
        (((1,), (1,)), ((), ())),
        preferred_element_type=jnp.float32,
    )


def _lang_tab(W_lang, W_proj):
    return pl.pallas_call(
        _lang_tab_body,
        out_shape=jax.ShapeDtypeStruct((NUM_LANG, D_MODEL), jnp.float32),
    )(W_lang, W_proj)


_mesh = plsc.VectorSubcoreMesh(core_axis_name="c", subcore_axis_name="s")


@functools.partial(
    pl.kernel,
    mesh=_mesh,
    out_type=jax.ShapeDtypeStruct((B * S, D_MODEL), jnp.float32),
    scratch_types=[
        pltpu.VMEM((_K,), jnp.int32),
        pltpu.VMEM((_K,), jnp.int32),
        pltpu.VMEM((_K, D_MODEL), jnp.float32),
        pltpu.VMEM((_K, D_MODEL), jnp.float32),
        pltpu.VMEM((NUM_LANG, D_MODEL), jnp.float32),
        pltpu.VMEM((_K, _L), jnp.int32),
        pltpu.SemaphoreType.DMA,
    ],
)
def _sc_embed(tok_ids, lang_ids, w_tok, lang_tab, pe, out,
              tok_idx_v, lang_idx_v, tokbuf, pebuf, lang_v, lidsplat, sem):
    cid = lax.axis_index("c")
    sid = lax.axis_index("s")
    wid = sid * 2 + cid
    s_base = wid * _SPW
    pltpu.sync_copy(lang_tab, lang_v)

    def chunk_loop(sc_i, _):
        s0 = s_base + sc_i * _K
        pltpu.sync_copy(pe.at[pl.ds(s0, _K)], pebuf)

        def b_loop(b, _):
            t0 = b * S + s0
            pltpu.sync_copy(tok_ids.at[pl.ds(t0, _K)], tok_idx_v)
            pltpu.sync_copy(lang_ids.at[pl.ds(t0, _K)], lang_idx_v)
            pltpu.async_copy(w_tok.at[tok_idx_v], tokbuf, sem).wait()

            def splat_grp(g, _):
                lvec = lang_idx_v[pl.ds(g * _L, _L)]
                for i16 in range(_L):
                    lidsplat.at[g * _L + i16][:] = _lane_splat(lvec, i16)
                return _

            lax.fori_loop(0, _K // _L, splat_grp, None)

            for jb in range(_NJ // _JB):
                rows = [
                    [lang_v.at[l][pl.ds((jb * _JB + j) * _L, _L)] for l in range(NUM_LANG)]
                    for j in range(_JB)
                ]

                def tok_loop(i, _, jb=jb, rows=rows):
                    lid = lidsplat.at[i][:]
                    m0 = lid == 0
                    m1 = lid == 1
                    m2 = lid == 2
                    for j in range(_JB):
                        jj = jb * _JB + j
                        t = tokbuf.at[i][pl.ds(jj * _L, _L)]
                        p = pebuf.at[i][pl.ds(jj * _L, _L)]
                        r = jnp.where(
                            m0, rows[j][0],
                            jnp.where(m1, rows[j][1],
                                      jnp.where(m2, rows[j][2], rows[j][3])),
                        )
                        tokbuf.at[i][pl.ds(jj * _L, _L)] = t * SCALE + p + r
                    return _

                lax.fori_loop(0, _K, tok_loop, None)

            pltpu.sync_copy(tokbuf, out.at[pl.ds(t0, _K)])
            return _

        lax.fori_loop(0, B, b_loop, None)
        return _

    lax.fori_loop(0, _NSC, chunk_loop, None)


def kernel(token_ids, lang_ids, W_tok, W_lang, W_proj):
    lang_tab = _lang_tab(W_lang, W_proj)
    tok_flat = token_ids.reshape(-1).astype(jnp.int32)
    lang_flat = lang_ids.reshape(-1).astype(jnp.int32)
    pe = jnp.asarray(_PE[:S])
    out = _sc_embed(tok_flat, lang_flat, W_tok, lang_tab, pe)
    return out.reshape(B, S, D_MODEL)
